# baseline (speedup 1.0000x reference)
import functools

import jax
import jax.numpy as jnp
from jax import lax
from jax.experimental import pallas as pl
from jax.experimental.pallas import tpu as pltpu
from jax.experimental.pallas import tpu_sc as plsc

_B, _H, _W, _C = 8, 224, 224, 384
_NROWS = _B * _H * _W
_ROW = 6 * _W + 6
_COL = 1
_NW = 32
_PERW = _NROWS // _NW
_CH = 128
_NIT = _PERW // (2 * _CH)
_TIT = _ROW // (2 * _CH)
_TBUF_OFF = _ROW - _TIT * 2 * _CH

_mesh = plsc.VectorSubcoreMesh(core_axis_name="c", subcore_axis_name="s")


@functools.partial(
    pl.kernel,
    out_type=jax.ShapeDtypeStruct((_NROWS, _C), jnp.float32),
    mesh=_mesh,
    scratch_types=[
        pltpu.VMEM((_CH, _C), jnp.float32),
        pltpu.VMEM((_CH, _C), jnp.float32),
        pltpu.SemaphoreType.DMA,
        pltpu.SemaphoreType.DMA,
        pltpu.SemaphoreType.DMA,
        pltpu.SemaphoreType.DMA,
    ],
)
def _sc_copy(x_hbm, o_hbm, buf0, buf1, si0, si1, so0, so1):
    wid = lax.axis_index("s") * 2 + lax.axis_index("c")
    base = wid * _PERW

    def body(i, _):
        c0 = base + i * (2 * _CH)
        c1 = c0 + _CH
        in0 = pltpu.make_async_copy(x_hbm.at[pl.ds(c0, _CH), :], buf0, si0)
        in0.start()
        in1 = pltpu.make_async_copy(x_hbm.at[pl.ds(c1, _CH), :], buf1, si1)
        in1.start()
        in0.wait()

        @pl.when(jnp.logical_and(wid == 0, i == _TIT))
        def _patch():
            v = buf0[_TBUF_OFF, pl.ds(0, 16)]
            lane = lax.broadcasted_iota(jnp.int32, (16,), 0)
            buf0[_TBUF_OFF, pl.ds(0, 16)] = jnp.where(
                lane == _COL, jnp.float32(1.0), v)

        out0 = pltpu.make_async_copy(buf0, o_hbm.at[pl.ds(c0, _CH), :], so0)
        out0.start()
        in1.wait()
        out1 = pltpu.make_async_copy(buf1, o_hbm.at[pl.ds(c1, _CH), :], so1)
        out1.start()
        out0.wait()
        out1.wait()
        return ()

    lax.fori_loop(0, _NIT, body, ())


def kernel(x):
    xf = x.reshape(_NROWS, _C)
    out = _sc_copy(xf)
    return out.reshape(_B, _H, _W, _C)


# speedup vs baseline: 1.0095x; 1.0095x over previous
import functools

import jax
import jax.numpy as jnp
from jax import lax
from jax.experimental import pallas as pl
from jax.experimental.pallas import tpu as pltpu
from jax.experimental.pallas import tpu_sc as plsc

_B, _H, _W, _C = 8, 224, 224, 384
_NROWS = _B * _H * _W
_ROW = 6 * _W + 6
_COL = 1
_NW = 32
_PERW = _NROWS // _NW
_NBUF = 4
_CH = 64
_G = _PERW // (_NBUF * _CH)
_TCHUNK = _ROW // _CH
_TG, _TSLOT = _TCHUNK // _NBUF, _TCHUNK % _NBUF
_TOFF = _ROW - _TCHUNK * _CH

_mesh = plsc.VectorSubcoreMesh(core_axis_name="c", subcore_axis_name="s")


@functools.partial(
    pl.kernel,
    out_type=jax.ShapeDtypeStruct((_NROWS, _C), jnp.float32),
    mesh=_mesh,
    scratch_types=(
        [pltpu.VMEM((_CH, _C), jnp.float32) for _ in range(_NBUF)]
        + [pltpu.SemaphoreType.DMA for _ in range(2 * _NBUF)]
    ),
)
def _sc_copy(x_hbm, o_hbm, *rest):
    bufs = rest[:_NBUF]
    in_sems = rest[_NBUF:2 * _NBUF]
    out_sems = rest[2 * _NBUF:3 * _NBUF]
    wid = lax.axis_index("s") * 2 + lax.axis_index("c")
    base = wid * _PERW

    def in_copy(rowstart, b):
        return pltpu.make_async_copy(
            x_hbm.at[pl.ds(rowstart, _CH), :], bufs[b], in_sems[b])

    def out_copy(rowstart, b):
        return pltpu.make_async_copy(
            bufs[b], o_hbm.at[pl.ds(rowstart, _CH), :], out_sems[b])

    for b in range(_NBUF):
        in_copy(base + b * _CH, b).start()

    def body(g, _):
        gbase = base + g * (_NBUF * _CH)
        for b in range(_NBUF):
            in_copy(gbase + b * _CH, b).wait()
            if b == _TSLOT:
                @pl.when(jnp.logical_and(wid == 0, g == _TG))
                def _patch():
                    v = bufs[_TSLOT][_TOFF, pl.ds(0, 16)]
                    lane = lax.broadcasted_iota(jnp.int32, (16,), 0)
                    bufs[_TSLOT][_TOFF, pl.ds(0, 16)] = jnp.where(
                        lane == _COL, jnp.float32(1.0), v)
            out_copy(gbase + b * _CH, b).start()
        nbase = gbase + _NBUF * _CH

        @pl.when(g < _G - 1)
        def _refill():
            for b in range(_NBUF):
                out_copy(gbase + b * _CH, b).wait()
                in_copy(nbase + b * _CH, b).start()
        return ()

    lax.fori_loop(0, _G, body, ())
    for b in range(_NBUF):
        out_copy(base + (_G - 1) * _NBUF * _CH + b * _CH, b).wait()


def kernel(x):
    xf = x.reshape(_NROWS, _C)
    out = _sc_copy(xf)
    return out.reshape(_B, _H, _W, _C)


# speedup vs baseline: 1.2053x; 1.1939x over previous
import jax
import jax.numpy as jnp
from jax.experimental import pallas as pl
from jax.experimental.pallas import tpu as pltpu

_B, _H, _W, _C = 8, 224, 224, 384
_NROWS = _B * _H * _W
_ROW = 6 * _W + 6
_COL = 1
_RING = 8
_LAG = 3
_CHUNK = 4096

_SIZES = [512, 1024, 2048] + [_CHUNK] * 96 + [1024, 2048, 1024, 512]
assert sum(_SIZES) == _NROWS
_STARTS = [sum(_SIZES[:i]) for i in range(len(_SIZES))]
_N = len(_SIZES)

_TCHUNK = next(i for i in range(_N)
               if _STARTS[i] <= _ROW < _STARTS[i] + _SIZES[i])
_TOFF = _ROW - _STARTS[_TCHUNK]
_TOFF8 = (_TOFF // 8) * 8


def _body(x_hbm, o_hbm, *rest):
    bufs = rest[:_RING]
    in_sems, out_sems = rest[_RING], rest[_RING + 1]
    in_copies = [None] * _N
    out_copies = [None] * _N
    out_waited = [False] * _N

    def start_in(i):
        b = i % _RING
        c = pltpu.make_async_copy(
            x_hbm.at[pl.ds(_STARTS[i], _SIZES[i]), :],
            bufs[b].at[pl.ds(0, _SIZES[i]), :], in_sems.at[b])
        c.start()
        in_copies[i] = c

    for i in range(min(_RING, _N)):
        start_in(i)
    for i in range(_N):
        b = i % _RING
        in_copies[i].wait()
        if i == _TCHUNK:
            r = jax.lax.broadcasted_iota(jnp.int32, (8, _C), 0)
            c2 = jax.lax.broadcasted_iota(jnp.int32, (8, _C), 1)
            hit = (r == (_TOFF - _TOFF8)) & (c2 == _COL)
            tile = bufs[b][pl.ds(_TOFF8, 8), :]
            bufs[b][pl.ds(_TOFF8, 8), :] = jnp.where(
                hit, jnp.float32(1.0), tile)
        oc = pltpu.make_async_copy(
            bufs[b].at[pl.ds(0, _SIZES[i]), :],
            o_hbm.at[pl.ds(_STARTS[i], _SIZES[i]), :], out_sems.at[b])
        oc.start()
        out_copies[i] = oc
        j = i - _LAG
        if j >= 0 and j + _RING < _N:
            out_copies[j].wait()
            out_waited[j] = True
            start_in(j + _RING)
    for i in range(_N):
        if not out_waited[i]:
            out_copies[i].wait()


def kernel(x):
    xf = x.reshape(_NROWS, _C)
    out = pl.pallas_call(
        _body,
        in_specs=[pl.BlockSpec(memory_space=pl.ANY)],
        out_specs=pl.BlockSpec(memory_space=pl.ANY),
        out_shape=jax.ShapeDtypeStruct((_NROWS, _C), jnp.float32),
        scratch_shapes=(
            [pltpu.VMEM((_CHUNK, _C), jnp.float32) for _ in range(_RING)]
            + [pltpu.SemaphoreType.DMA((_RING,)),
               pltpu.SemaphoreType.DMA((_RING,))]
        ),
    )(xf)
    return out.reshape(_B, _H, _W, _C)
